# speedup vs baseline: 2.7691x; 2.7691x over previous
import functools

import jax
import jax.numpy as jnp
from jax import lax
from jax.experimental import pallas as pl
from jax.experimental.pallas import tpu as pltpu
from jax.experimental.pallas import tpu_sc as plsc

NUM_CORES = 2
NUM_SUBCORES = 16
NUM_WORKERS = NUM_CORES * NUM_SUBCORES
CHUNK = 128


def _gather_rows(table, idx):
    B = idx.shape[0]
    D = table.shape[1]
    b_per_w = B // NUM_WORKERS
    n_chunks = b_per_w // CHUNK
    mesh = plsc.VectorSubcoreMesh(core_axis_name="c", subcore_axis_name="s")

    @functools.partial(
        pl.kernel,
        mesh=mesh,
        out_type=jax.ShapeDtypeStruct((B, D), jnp.float32),
        scratch_types=[
            pltpu.VMEM((CHUNK,), jnp.int32),
            pltpu.VMEM((CHUNK, D), jnp.float32),
            pltpu.SemaphoreType.DMA,
        ],
    )
    def k(idx_hbm, table_hbm, out_hbm, idx_v, rows_v, sem):
        wid = lax.axis_index("s") * NUM_CORES + lax.axis_index("c")
        base = wid * b_per_w

        @pl.loop(0, n_chunks)
        def _(ci):
            off = base + ci * CHUNK
            pltpu.sync_copy(idx_hbm.at[pl.ds(off, CHUNK)], idx_v)
            pltpu.async_copy(table_hbm.at[idx_v], rows_v, sem).wait()
            pltpu.sync_copy(rows_v, out_hbm.at[pl.ds(off, CHUNK)])

    return k(idx, table)


@jax.jit
def kernel(x, weight):
    B = x.shape[0] * x.shape[1]
    idx = x.reshape(B).astype(jnp.int32)
    out = _gather_rows(weight, idx)
    return out.reshape(x.shape[0], x.shape[1], weight.shape[1])


# speedup vs baseline: 3.2054x; 1.1575x over previous
import functools

import jax
import jax.numpy as jnp
from jax import lax
from jax.experimental import pallas as pl
from jax.experimental.pallas import tpu as pltpu
from jax.experimental.pallas import tpu_sc as plsc

NUM_CORES = 2
NUM_SUBCORES = 16
NUM_WORKERS = NUM_CORES * NUM_SUBCORES
CHUNK = 64
NBUF = 4


def _gather_rows(table, idx):
    B = idx.shape[0]
    D = table.shape[1]
    b_per_w = B // NUM_WORKERS
    n_chunks = b_per_w // CHUNK
    n_rounds = n_chunks // NBUF
    idx3d = idx.reshape(NUM_WORKERS, b_per_w // CHUNK, CHUNK)
    mesh = plsc.VectorSubcoreMesh(core_axis_name="c", subcore_axis_name="s")

    @functools.partial(
        pl.kernel,
        mesh=mesh,
        out_type=jax.ShapeDtypeStruct((B, D), jnp.float32),
        scratch_types=[
            pltpu.VMEM((n_chunks, CHUNK), jnp.int32),
            pltpu.VMEM((NBUF, CHUNK, D), jnp.float32),
            pltpu.SemaphoreType.DMA,
            pltpu.SemaphoreType.DMA,
            pltpu.SemaphoreType.DMA,
            pltpu.SemaphoreType.DMA,
            pltpu.SemaphoreType.DMA,
            pltpu.SemaphoreType.DMA,
            pltpu.SemaphoreType.DMA,
            pltpu.SemaphoreType.DMA,
        ],
    )
    def k(idx_hbm, table_hbm, out_hbm, idx_v, rows, g0, g1, g2, g3, o0, o1, o2, o3):
        gsem = (g0, g1, g2, g3)
        osem = (o0, o1, o2, o3)
        wid = lax.axis_index("s") * NUM_CORES + lax.axis_index("c")
        base = wid * b_per_w

        pltpu.sync_copy(idx_hbm.at[wid], idx_v)

        @pl.loop(0, n_rounds)
        def _(r):
            c0 = r * NBUF
            @pl.when(r > 0)
            def _():
                for b in range(NBUF):
                    pltpu.make_async_copy(
                        rows.at[b], out_hbm.at[pl.ds(base, CHUNK)],
                        osem[b]).wait()

            gathers = [
                pltpu.async_copy(table_hbm.at[idx_v.at[c0 + b]], rows.at[b],
                                 gsem[b])
                for b in range(NBUF)
            ]
            for b in range(NBUF):
                gathers[b].wait()
                pltpu.async_copy(
                    rows.at[b],
                    out_hbm.at[pl.ds(base + (c0 + b) * CHUNK, CHUNK)],
                    osem[b])

        for b in range(NBUF):
            pltpu.make_async_copy(rows.at[b], out_hbm.at[pl.ds(base, CHUNK)],
                                  osem[b]).wait()

    return k(idx3d, table)


@jax.jit
def kernel(x, weight):
    B = x.shape[0] * x.shape[1]
    idx = x.reshape(B).astype(jnp.int32)
    out = _gather_rows(weight, idx)
    return out.reshape(x.shape[0], x.shape[1], weight.shape[1])


# speedup vs baseline: 5.7410x; 1.7911x over previous
import functools

import jax
import jax.numpy as jnp
from jax import lax
from jax.experimental import pallas as pl
from jax.experimental.pallas import tpu as pltpu
from jax.experimental.pallas import tpu_sc as plsc

NUM_CORES = 2
NUM_SUBCORES = 16
NUM_WORKERS = NUM_CORES * NUM_SUBCORES
ROWS_PER_CHUNK = 2
NBUF = 4


def _gather_rows(table, idx3d, n_batch, seq):
    D = table.shape[1]
    rows_per_w = n_batch // NUM_WORKERS
    n_chunks = rows_per_w // ROWS_PER_CHUNK
    n_rounds = n_chunks // NBUF
    chunk_idx = ROWS_PER_CHUNK * seq
    mesh = plsc.VectorSubcoreMesh(core_axis_name="c", subcore_axis_name="s")

    @functools.partial(
        pl.kernel,
        mesh=mesh,
        out_type=jax.ShapeDtypeStruct((n_batch, seq, D), jnp.float32),
        scratch_types=[
            pltpu.VMEM((n_chunks, chunk_idx), jnp.int32),
            pltpu.VMEM((NBUF, chunk_idx, D), jnp.float32),
            pltpu.SemaphoreType.DMA,
            pltpu.SemaphoreType.DMA,
            pltpu.SemaphoreType.DMA,
            pltpu.SemaphoreType.DMA,
            pltpu.SemaphoreType.DMA,
            pltpu.SemaphoreType.DMA,
            pltpu.SemaphoreType.DMA,
            pltpu.SemaphoreType.DMA,
        ],
    )
    def k(idx_hbm, table_hbm, out_hbm, idx_v, rows, g0, g1, g2, g3, o0, o1, o2, o3):
        gsem = (g0, g1, g2, g3)
        osem = (o0, o1, o2, o3)
        wid = lax.axis_index("s") * NUM_CORES + lax.axis_index("c")
        base_b = wid * rows_per_w

        pltpu.sync_copy(idx_hbm.at[wid], idx_v)

        @pl.loop(0, n_rounds)
        def _(r):
            c0 = r * NBUF
            @pl.when(r > 0)
            def _():
                for b in range(NBUF):
                    for _h in range(ROWS_PER_CHUNK):
                        pltpu.make_async_copy(
                            rows.at[b, pl.ds(0, seq)], out_hbm.at[base_b],
                            osem[b]).wait()

            gathers = [
                pltpu.async_copy(table_hbm.at[idx_v.at[c0 + b]], rows.at[b],
                                 gsem[b])
                for b in range(NBUF)
            ]
            for b in range(NBUF):
                gathers[b].wait()
                c = c0 + b
                for h in range(ROWS_PER_CHUNK):
                    pltpu.async_copy(
                        rows.at[b, pl.ds(h * seq, seq)],
                        out_hbm.at[base_b + ROWS_PER_CHUNK * c + h],
                        osem[b])

        for b in range(NBUF):
            for _h in range(ROWS_PER_CHUNK):
                pltpu.make_async_copy(rows.at[b, pl.ds(0, seq)],
                                      out_hbm.at[base_b], osem[b]).wait()

    return k(idx3d, table)


@jax.jit
def kernel(x, weight):
    n_batch, seq = x.shape
    rows_per_w = n_batch // NUM_WORKERS
    idx3d = x.astype(jnp.int32).reshape(
        NUM_WORKERS, rows_per_w // ROWS_PER_CHUNK, ROWS_PER_CHUNK * seq)
    return _gather_rows(weight, idx3d, n_batch, seq)


# speedup vs baseline: 10.3615x; 1.8048x over previous
import functools

import jax
import jax.numpy as jnp
from jax import lax
from jax.experimental import pallas as pl
from jax.experimental.pallas import tpu as pltpu
from jax.experimental.pallas import tpu_sc as plsc

NUM_CORES = 2
NUM_SUBCORES = 16
NUM_WORKERS = NUM_CORES * NUM_SUBCORES
NBUF = 5


def _gather_rows(table, idx_t, n_batch, seq):
    D = table.shape[1]
    bw = n_batch // NUM_WORKERS
    n_rounds = seq // NBUF
    mesh = plsc.VectorSubcoreMesh(core_axis_name="c", subcore_axis_name="s")

    @functools.partial(
        pl.kernel,
        mesh=mesh,
        out_type=jax.ShapeDtypeStruct((seq, n_batch, D), jnp.float32),
        scratch_types=[
            pltpu.VMEM((seq, bw), jnp.int32),
            pltpu.VMEM((NBUF, bw, D), jnp.float32),
            pltpu.SemaphoreType.DMA,
            pltpu.SemaphoreType.DMA,
            pltpu.SemaphoreType.DMA,
            pltpu.SemaphoreType.DMA,
            pltpu.SemaphoreType.DMA,
            pltpu.SemaphoreType.DMA,
            pltpu.SemaphoreType.DMA,
            pltpu.SemaphoreType.DMA,
            pltpu.SemaphoreType.DMA,
            pltpu.SemaphoreType.DMA,
        ],
    )
    def k(idx_hbm, table_hbm, out_hbm, idx_v, rows,
          g0, g1, g2, g3, g4, o0, o1, o2, o3, o4):
        gsem = (g0, g1, g2, g3, g4)
        osem = (o0, o1, o2, o3, o4)
        wid = lax.axis_index("s") * NUM_CORES + lax.axis_index("c")
        base_b = wid * bw

        pltpu.sync_copy(idx_hbm.at[:, pl.ds(base_b, bw)], idx_v)

        @pl.loop(0, n_rounds)
        def _(r):
            s0 = r * NBUF
            @pl.when(r > 0)
            def _():
                for b in range(NBUF):
                    pltpu.make_async_copy(
                        rows.at[b], out_hbm.at[0, pl.ds(base_b, bw)],
                        osem[b]).wait()

            gathers = [
                pltpu.async_copy(table_hbm.at[idx_v.at[s0 + b]], rows.at[b],
                                 gsem[b])
                for b in range(NBUF)
            ]
            for b in range(NBUF):
                gathers[b].wait()
                pltpu.async_copy(
                    rows.at[b], out_hbm.at[s0 + b, pl.ds(base_b, bw)],
                    osem[b])

        for b in range(NBUF):
            pltpu.make_async_copy(rows.at[b], out_hbm.at[0, pl.ds(base_b, bw)],
                                  osem[b]).wait()

    return k(idx_t, table)


@jax.jit
def kernel(x, weight):
    n_batch, seq = x.shape
    idx_t = x.astype(jnp.int32).T
    out_t = _gather_rows(weight, idx_t, n_batch, seq)
    return out_t.transpose(1, 0, 2)


# speedup vs baseline: 10.4252x; 1.0061x over previous
import functools

import jax
import jax.numpy as jnp
from jax import lax
from jax.experimental import pallas as pl
from jax.experimental.pallas import tpu as pltpu
from jax.experimental.pallas import tpu_sc as plsc

NUM_CORES = 2
NUM_SUBCORES = 16
NUM_WORKERS = NUM_CORES * NUM_SUBCORES
NBUF = 5


def _gather_rows(table, idx_t, n_batch, seq):
    D = table.shape[1]
    bw = n_batch // NUM_WORKERS
    n_rounds = seq // NBUF
    mesh = plsc.VectorSubcoreMesh(core_axis_name="c", subcore_axis_name="s")

    @functools.partial(
        pl.kernel,
        mesh=mesh,
        out_type=jax.ShapeDtypeStruct((seq, n_batch, D), jnp.float32),
        scratch_types=[
            pltpu.VMEM((seq, bw), jnp.int32),
            pltpu.VMEM((NBUF, bw, D), jnp.float32),
            pltpu.SemaphoreType.DMA,
            pltpu.SemaphoreType.DMA,
            pltpu.SemaphoreType.DMA,
            pltpu.SemaphoreType.DMA,
            pltpu.SemaphoreType.DMA,
            pltpu.SemaphoreType.DMA,
            pltpu.SemaphoreType.DMA,
            pltpu.SemaphoreType.DMA,
            pltpu.SemaphoreType.DMA,
            pltpu.SemaphoreType.DMA,
        ],
    )
    def k(idx_hbm, table_hbm, out_hbm, idx_v, rows,
          g0, g1, g2, g3, g4, o0, o1, o2, o3, o4):
        gsem = (g0, g1, g2, g3, g4)
        osem = (o0, o1, o2, o3, o4)
        wid = lax.axis_index("s") * NUM_CORES + lax.axis_index("c")
        base_b = wid * bw

        pltpu.sync_copy(idx_hbm.at[:, pl.ds(base_b, bw)], idx_v)

        @pl.loop(0, n_rounds)
        def _(r):
            s0 = r * NBUF
            gathers = []
            for b in range(NBUF):
                @pl.when(r > 0)
                def _(b=b):
                    pltpu.make_async_copy(
                        rows.at[b], out_hbm.at[0, pl.ds(base_b, bw)],
                        osem[b]).wait()
                gathers.append(
                    pltpu.async_copy(table_hbm.at[idx_v.at[s0 + b]],
                                     rows.at[b], gsem[b]))
            for b in range(NBUF):
                gathers[b].wait()
                pltpu.async_copy(
                    rows.at[b], out_hbm.at[s0 + b, pl.ds(base_b, bw)],
                    osem[b])

        for b in range(NBUF):
            pltpu.make_async_copy(rows.at[b], out_hbm.at[0, pl.ds(base_b, bw)],
                                  osem[b]).wait()

    return k(idx_t, table)


@jax.jit
def kernel(x, weight):
    n_batch, seq = x.shape
    idx_t = x.astype(jnp.int32).T
    out_t = _gather_rows(weight, idx_t, n_batch, seq)
    return out_t.transpose(1, 0, 2)
